# baseline (speedup 1.0000x reference)
import functools

import jax
import jax.numpy as jnp
from jax import lax
from jax.experimental import pallas as pl
from jax.experimental.pallas import tpu as pltpu

_LANES = 128
_SUB = 8
_BLOCK_ROWS = 16384
_CHUNK_ROWS = 512


def _block_sq_sum(x_ref, y_ref, *, br, chunk, base_row, rows, masked):
    nchunks = br // chunk

    def body(c, carry):
        r0 = pl.multiple_of(c * chunk, chunk)
        d = (x_ref[pl.ds(r0, chunk), :].astype(jnp.float32)
             - y_ref[pl.ds(r0, chunk), :].astype(jnp.float32))
        dd = d * d
        if masked:
            row = base_row + r0 + lax.broadcasted_iota(jnp.int32, (chunk, 1), 0)
            dd = jnp.where(row < rows, dd, 0.0)
        return carry + dd.reshape(chunk // _SUB, _SUB, _LANES).sum(axis=0)

    return lax.fori_loop(0, nchunks, body,
                         jnp.zeros((_SUB, _LANES), jnp.float32),
                         unroll=nchunks <= 8)


def _mse_kernel(x_ref, y_ref, o_ref, acc_ref, *, br, chunk, rows, needs_mask,
                scale):
    k = pl.program_id(0)
    nblk = pl.num_programs(0)

    @pl.when(k == 0)
    def _():
        acc_ref[...] = jnp.zeros_like(acc_ref)

    base_row = k * br
    if needs_mask:
        @pl.when(k == nblk - 1)
        def _():
            acc_ref[...] += _block_sq_sum(x_ref, y_ref, br=br, chunk=chunk,
                                          base_row=base_row, rows=rows,
                                          masked=True)

        @pl.when(k < nblk - 1)
        def _():
            acc_ref[...] += _block_sq_sum(x_ref, y_ref, br=br, chunk=chunk,
                                          base_row=base_row, rows=rows,
                                          masked=False)
    else:
        acc_ref[...] += _block_sq_sum(x_ref, y_ref, br=br, chunk=chunk,
                                      base_row=base_row, rows=rows,
                                      masked=False)

    @pl.when(k == nblk - 1)
    def _():
        o_ref[0] = jnp.sum(acc_ref[...]) * scale


def _mse_mean_slab(x2, y2, rows, n):
    itemsize = jnp.dtype(x2.dtype).itemsize
    br = min(_BLOCK_ROWS, max(_SUB, (rows // _SUB) * _SUB))
    chunk = min(br, _CHUNK_ROWS)
    br = max(chunk, (br // chunk) * chunk)
    num_blocks = pl.cdiv(rows, br)
    needs_mask = num_blocks * br != rows

    tile_bytes = br * _LANES * itemsize
    vmem_limit = int(min(int((64 << 20) * 0.7),
                         max(16 << 20, 2 * 2 * tile_bytes + (4 << 20))))

    kernel_fn = functools.partial(_mse_kernel, br=br, chunk=chunk, rows=rows,
                                  needs_mask=needs_mask, scale=1.0 / n)
    out = pl.pallas_call(
        kernel_fn,
        out_shape=jax.ShapeDtypeStruct((1,), jnp.float32),
        grid_spec=pltpu.PrefetchScalarGridSpec(
            num_scalar_prefetch=0,
            grid=(num_blocks,),
            in_specs=[pl.BlockSpec((br, _LANES), lambda k: (k, 0)),
                      pl.BlockSpec((br, _LANES), lambda k: (k, 0))],
            out_specs=pl.BlockSpec(memory_space=pltpu.MemorySpace.SMEM),
            scratch_shapes=[pltpu.VMEM((_SUB, _LANES), jnp.float32)],
        ),
        compiler_params=pltpu.CompilerParams(
            dimension_semantics=("arbitrary",),
            vmem_limit_bytes=vmem_limit),
        cost_estimate=pl.CostEstimate(
            flops=3 * rows * _LANES, transcendentals=0,
            bytes_accessed=2 * rows * _LANES * itemsize + 1024),
    )(x2, y2)
    return out[0]


def kernel(x, y):
    n = x.size
    xf = jnp.ravel(x)
    yf = jnp.ravel(y)
    main = (n // _LANES) * _LANES
    rows = main // _LANES
    if rows < _SUB:
        main, rows = 0, 0

    total = jnp.float32(0.0)
    if rows > 0:
        if main == n:
            x2 = xf.reshape(rows, _LANES)
            y2 = yf.reshape(rows, _LANES)
        else:
            x2 = lax.slice(xf, (0,), (main,)).reshape(rows, _LANES)
            y2 = lax.slice(yf, (0,), (main,)).reshape(rows, _LANES)
        total = total + _mse_mean_slab(x2, y2, rows, n)
    if main != n:
        xt = lax.slice(xf, (main,), (n,)).astype(jnp.float32)
        yt = lax.slice(yf, (main,), (n,)).astype(jnp.float32)
        d = xt - yt
        total = total + jnp.sum(d * d) / n
    return total.astype(x.dtype)


# speedup vs baseline: 4.5634x; 4.5634x over previous
import functools

import jax
import jax.numpy as jnp
from jax import lax
from jax.experimental import pallas as pl
from jax.experimental.pallas import tpu as pltpu

_LANES = 128
_SUB = 8
_BLOCK_ROWS = 256
_CHUNK_ROWS = 32


def _block_sq_sum(x_ref, y_ref, *, br, chunk, cols, base_row, rows, masked):
    nchunks = br // chunk
    tiles = cols // _LANES

    def body(c, carry):
        r0 = pl.multiple_of(c * chunk, chunk)
        d = (x_ref[pl.ds(r0, chunk), :].astype(jnp.float32)
             - y_ref[pl.ds(r0, chunk), :].astype(jnp.float32))
        dd = d * d
        if masked:
            row = base_row + r0 + lax.broadcasted_iota(jnp.int32, (chunk, 1), 0)
            dd = jnp.where(row < rows, dd, 0.0)
        return carry + dd.reshape(chunk // _SUB, _SUB, tiles, _LANES).sum(
            axis=(0, 2))

    return lax.fori_loop(0, nchunks, body,
                         jnp.zeros((_SUB, _LANES), jnp.float32),
                         unroll=nchunks <= 8)


def _mse_kernel(x_ref, y_ref, o_ref, acc_ref, *, br, chunk, cols, rows,
                needs_mask, scale):
    k = pl.program_id(0)
    nblk = pl.num_programs(0)

    @pl.when(k == 0)
    def _():
        acc_ref[...] = jnp.zeros_like(acc_ref)

    base_row = k * br
    if needs_mask:
        @pl.when(k == nblk - 1)
        def _():
            acc_ref[...] += _block_sq_sum(x_ref, y_ref, br=br, chunk=chunk,
                                          cols=cols, base_row=base_row,
                                          rows=rows, masked=True)

        @pl.when(k < nblk - 1)
        def _():
            acc_ref[...] += _block_sq_sum(x_ref, y_ref, br=br, chunk=chunk,
                                          cols=cols, base_row=base_row,
                                          rows=rows, masked=False)
    else:
        acc_ref[...] += _block_sq_sum(x_ref, y_ref, br=br, chunk=chunk,
                                      cols=cols, base_row=base_row,
                                      rows=rows, masked=False)

    @pl.when(k == nblk - 1)
    def _():
        o_ref[0] = jnp.sum(acc_ref[...]) * scale


def _mse_mean_2d(x, y, rows, cols, n):
    itemsize = jnp.dtype(x.dtype).itemsize
    br = min(_BLOCK_ROWS, max(_SUB, (rows // _SUB) * _SUB))
    chunk = min(br, _CHUNK_ROWS)
    br = max(chunk, (br // chunk) * chunk)
    num_blocks = pl.cdiv(rows, br)
    needs_mask = num_blocks * br != rows

    tile_bytes = br * cols * itemsize
    vmem_limit = int(min(int((64 << 20) * 0.7),
                         max(16 << 20, 2 * 2 * tile_bytes + (4 << 20))))

    kernel_fn = functools.partial(_mse_kernel, br=br, chunk=chunk, cols=cols,
                                  rows=rows, needs_mask=needs_mask,
                                  scale=1.0 / n)
    out = pl.pallas_call(
        kernel_fn,
        out_shape=jax.ShapeDtypeStruct((1,), jnp.float32),
        grid_spec=pltpu.PrefetchScalarGridSpec(
            num_scalar_prefetch=0,
            grid=(num_blocks,),
            in_specs=[pl.BlockSpec((br, cols), lambda k: (k, 0)),
                      pl.BlockSpec((br, cols), lambda k: (k, 0))],
            out_specs=pl.BlockSpec(memory_space=pltpu.MemorySpace.SMEM),
            scratch_shapes=[pltpu.VMEM((_SUB, _LANES), jnp.float32)],
        ),
        compiler_params=pltpu.CompilerParams(
            dimension_semantics=("arbitrary",),
            vmem_limit_bytes=vmem_limit),
        cost_estimate=pl.CostEstimate(
            flops=3 * rows * cols, transcendentals=0,
            bytes_accessed=2 * rows * cols * itemsize + 1024),
    )(x, y)
    return out[0]


def kernel(x, y):
    n = x.size
    if (x.ndim == 2 and x.shape[1] % _LANES == 0 and x.shape[0] >= _SUB
            and x.shape[1] > 0):
        return _mse_mean_2d(x, y, x.shape[0], x.shape[1], n).astype(x.dtype)

    xf = jnp.ravel(x)
    yf = jnp.ravel(y)
    main = (n // _LANES) * _LANES
    rows = main // _LANES
    if rows < _SUB:
        main, rows = 0, 0

    total = jnp.float32(0.0)
    if rows > 0:
        x2 = lax.slice(xf, (0,), (main,)).reshape(rows, _LANES)
        y2 = lax.slice(yf, (0,), (main,)).reshape(rows, _LANES)
        total = total + _mse_mean_2d(x2, y2, rows, _LANES, n)
    if main != n:
        xt = lax.slice(xf, (main,), (n,)).astype(jnp.float32)
        yt = lax.slice(yf, (main,), (n,)).astype(jnp.float32)
        d = xt - yt
        total = total + jnp.sum(d * d) / n
    return total.astype(x.dtype)
